# Initial kernel scaffold; baseline (speedup 1.0000x reference)
import jax
import jax.numpy as jnp
from jax.experimental import pallas as pl


def kernel(x, pos, word_table, pos_table):
    raise NotImplementedError("write your pallas kernel here")


# speedup vs baseline: 2.0416x; 2.0416x over previous
import functools

import jax
import jax.numpy as jnp
from jax import lax
from jax.experimental import pallas as pl
from jax.experimental.pallas import tpu as pltpu
from jax.experimental.pallas import tpu_sc as plsc

_B, _S = 4096, 200
_N = _B * _S
_WD, _PD, _OD = 60, 4, 64
_MAXLEN = 200
_NC, _NS = 2, 16
_NW = _NC * _NS
_ROWS_W = _N // _NW
_IB = 128
_K = 8
_CH = _IB * _K
_NCH = _ROWS_W // _CH
_L = 16
_TC0 = _OD - _L


def _emb_body(x_hbm, p_hbm, wt_hbm, pt_hbm, out_hbm,
              xidx, pidx_v, out_v, pt_v, tail_v, sem_i, sem_w):
    sid = lax.axis_index("s")
    wid = sid * _NC + lax.axis_index("c")
    row0 = wid * _ROWS_W

    pltpu.sync_copy(pt_hbm, pt_v)

    def chunk(ci, carry):
        base = pl.multiple_of(row0 + ci * _CH, _CH)
        ib = pl.multiple_of(base // _IB, 8)
        cp_x = pltpu.async_copy(x_hbm.at[pl.ds(ib, _K)], xidx, sem_i)
        cp_p = pltpu.async_copy(p_hbm.at[pl.ds(base, _CH)], pidx_v, sem_i)
        cp_x.wait()
        cp_p.wait()
        ws = [pltpu.async_copy(wt_hbm.at[xidx.at[j]],
                               out_v.at[pl.ds(j * _IB, _IB)], sem_w)
              for j in range(_K)]
        for c in ws:
            c.wait()

        def tail_fill(g, carry2):
            prow = pidx_v[pl.ds(g * _L, _L)]
            for u in range(_L):
                r = g * _L + u
                wvec = out_v[r, pl.ds(_TC0, _L)]
                pvec = pt_v[prow[u], :]
                tail_v[r, :] = wvec + pvec
            return carry2

        lax.fori_loop(0, _CH // _L, tail_fill, 0)
        pltpu.sync_copy(out_v, out_hbm.at[pl.ds(base, _CH)])
        pltpu.sync_copy(tail_v, out_hbm.at[pl.ds(base, _CH), pl.ds(_TC0, _L)])
        return carry

    lax.fori_loop(0, _NCH, chunk, 0)


_emb = pl.kernel(
    _emb_body,
    out_type=jax.ShapeDtypeStruct((_N, _OD), jnp.float32),
    mesh=plsc.VectorSubcoreMesh(core_axis_name="c", subcore_axis_name="s"),
    scratch_types=[
        pltpu.VMEM((_K, _IB), jnp.int32),
        pltpu.VMEM((_CH,), jnp.int32),
        pltpu.VMEM((_CH, _OD), jnp.float32),
        pltpu.VMEM((_MAXLEN, _L), jnp.float32),
        pltpu.VMEM((_CH, _L), jnp.float32),
        pltpu.SemaphoreType.DMA,
        pltpu.SemaphoreType.DMA,
    ],
    compiler_params=pltpu.CompilerParams(use_tc_tiling_on_sc=False),
)


def kernel(x, pos, word_table, pos_table):
    wt64 = jnp.pad(word_table, ((0, 0), (0, _OD - _WD)))
    pt16 = jnp.pad(pos_table, ((0, 0), (_L - _PD, 0)))
    xf = x.reshape(_N // _IB, _IB)
    pf = pos.reshape(_N)
    out = _emb(xf, pf, wt64, pt16)
    return out.reshape(_B, _S, _OD)
